# baseline (speedup 1.0000x reference)
import functools

import jax
import jax.numpy as jnp
from jax import lax
from jax.experimental import pallas as pl
from jax.experimental.pallas import tpu as pltpu
from jax.experimental.pallas import tpu_sc as plsc

N = 10000
E = 160000
D = 256
H1 = 4
C1 = 256
C2 = 256

NC = 2
NS = 16
NW = NC * NS
EPT = 5120
EPAD = NW * EPT
KW = 256
WPT = EPT // KW
VPT = EPT // 16
RPT = N // NS
CW = 64

_MESH = dict(core_axis_name="c", subcore_axis_name="s")
_PARAMS = dict(compiler_params=pltpu.CompilerParams(
    needs_layout_passes=False, use_tc_tiling_on_sc=False))


def _sc_edge(H, NHP):
    ZS = NHP // NS
    KD = 512
    NWD = EPT // KD

    @functools.partial(
        pl.kernel,
        out_type=(jax.ShapeDtypeStruct((H, EPAD), jnp.float32),
                  jax.ShapeDtypeStruct((NC, NHP), jnp.float32)),
        mesh=plsc.VectorSubcoreMesh(**_MESH),
        scratch_types=[
            pltpu.VMEM((EPT,), jnp.int32),
            pltpu.VMEM((EPT,), jnp.int32),
            pltpu.VMEM((H, EPT), jnp.float32),
            pltpu.VMEM((KD,), jnp.int32),
            pltpu.VMEM((KD, 16), jnp.float32),
            pltpu.VMEM((KD, 16), jnp.float32),
            pltpu.VMEM((KD, 16), jnp.float32),
            pltpu.VMEM((128, 16), jnp.float32),
            pltpu.VMEM((128,), jnp.float32),
            pltpu.SemaphoreType.DMA,
            pltpu.VMEM_SHARED((N, 16), jnp.float32),
            pltpu.VMEM_SHARED((NHP, 16), jnp.float32),
        ],
        **_PARAMS,
    )
    def k(a16_hbm, src_hbm, dst_hbm, ex_out, den_out,
          srcv, dstv, exv, didxb, exrows, arows_s, arows_d, cbuf, compb, sem,
          a16sp, den2d):
        cid = lax.axis_index("c")
        sid = lax.axis_index("s")
        w = cid * NS + sid
        lane = lax.iota(jnp.int32, 16)
        zeros16i = jnp.zeros((16,), jnp.int32)

        def zr(i, c):
            exrows[i, pl.ds(0, 16)] = jnp.zeros((16,), jnp.float32)
            return c
        lax.fori_loop(0, KD, zr, 0)
        for b in range(ZS // KD):
            pltpu.sync_copy(exrows, den2d.at[pl.ds(sid * ZS + b * KD, KD)])
        if ZS % KD:
            pltpu.sync_copy(exrows.at[pl.ds(0, ZS % KD)],
                            den2d.at[pl.ds(sid * ZS + (ZS // KD) * KD, ZS % KD)])
        nb0 = sid * 624
        pltpu.sync_copy(a16_hbm.at[pl.ds(nb0, KD)], arows_s)
        pltpu.sync_copy(arows_s, a16sp.at[pl.ds(nb0, KD)])
        pltpu.sync_copy(a16_hbm.at[pl.ds(nb0 + KD, 128)],
                        arows_s.at[pl.ds(0, 128)])
        pltpu.sync_copy(arows_s.at[pl.ds(0, 128)],
                        a16sp.at[pl.ds(nb0 + KD, 128)])
        pltpu.sync_copy(src_hbm.at[w], srcv)
        pltpu.sync_copy(dst_hbm.at[w], dstv)
        plsc.subcore_barrier()

        ebase = w * EPT

        def wloop(wi, c):
            off0 = wi * KD
            pltpu.async_copy(a16sp.at[srcv.at[pl.ds(off0, KD)]],
                             arows_s, sem).wait()
            pltpu.async_copy(a16sp.at[dstv.at[pl.ds(off0, KD)]],
                             arows_d, sem).wait()

            def body(i, c2):
                valid = (ebase + off0 + i * 16 + lane) < E
                rows = lane + i * 16
                for h in range(H):
                    av = plsc.load_gather(arows_s, [rows, zeros16i + h])
                    bv = plsc.load_gather(arows_d, [rows, zeros16i + (4 + h)])
                    e = av + bv
                    e = jnp.maximum(e, 0.2 * e)
                    ex = jnp.where(valid, jnp.exp(e), 0.0)
                    exv[h, pl.ds(off0 + i * 16, 16)] = ex
                return c2
            lax.fori_loop(0, KD // 16, body, 0)

            for h in range(H):
                def fill(i, c2, _h=h):
                    d16 = dstv[pl.ds(off0 + i * 16, 16)]
                    didxb[pl.ds(i * 16, 16)] = d16 * H + _h
                    ex16 = exv[_h, pl.ds(off0 + i * 16, 16)]
                    plsc.store_scatter(exrows, [lane + i * 16, zeros16i], ex16)
                    return c2
                lax.fori_loop(0, KD // 16, fill, 0)
                pltpu.sync_copy(exrows, den2d.at[didxb], add=True)
            return c
        lax.fori_loop(0, NWD, wloop, 0)

        for h in range(H):
            pltpu.sync_copy(exv.at[h], ex_out.at[h, pl.ds(ebase, EPT)])

        plsc.subcore_barrier()

        def comp(t, c):
            base = sid * ZS + t * 128
            pltpu.sync_copy(den2d.at[pl.ds(base, 128)], cbuf)

            def gat(j, c2):
                g = plsc.load_gather(cbuf, [lane + j * 16, zeros16i])
                compb[pl.ds(j * 16, 16)] = g
                return c2
            lax.fori_loop(0, 8, gat, 0)
            pltpu.sync_copy(compb, den_out.at[cid, pl.ds(base, 128)])
            return c
        lax.fori_loop(0, ZS // 128, comp, 0)

    return k


def _sc_alpha(H, NHP):
    @functools.partial(
        pl.kernel,
        out_type=jax.ShapeDtypeStruct((H, EPAD), jnp.float32),
        mesh=plsc.VectorSubcoreMesh(**_MESH),
        scratch_types=[
            pltpu.VMEM((NHP,), jnp.float32),
            pltpu.VMEM((2048,), jnp.float32),
            pltpu.VMEM((EPT,), jnp.int32),
            pltpu.VMEM((EPT,), jnp.float32),
            pltpu.VMEM((EPT,), jnp.float32),
        ],
        **_PARAMS,
    )
    def k(ex_hbm, dst_hbm, den_hbm, alpha_out, denl, tmp, dstv, exv, av):
        cid = lax.axis_index("c")
        sid = lax.axis_index("s")
        w = cid * NS + sid

        pltpu.sync_copy(den_hbm.at[0], denl)

        def mbody(kk, c):
            pltpu.sync_copy(den_hbm.at[1, pl.ds(kk * 2048, 2048)], tmp)

            def abody(j, c2):
                sl = pl.ds(kk * 2048 + j * 16, 16)
                denl[sl] = denl[sl] + tmp[pl.ds(j * 16, 16)] + 1e-16
                return c2
            lax.fori_loop(0, 128, abody, 0)
            return c
        lax.fori_loop(0, NHP // 2048, mbody, 0)

        pltpu.sync_copy(dst_hbm.at[w], dstv)
        for h in range(H):
            pltpu.sync_copy(ex_hbm.at[h, pl.ds(w * EPT, EPT)], exv)

            def body(i, c, _h=h):
                d16 = dstv[pl.ds(i * 16, 16)]
                dv = plsc.load_gather(denl, [d16 * H + _h])
                av[pl.ds(i * 16, 16)] = exv[pl.ds(i * 16, 16)] / dv
                return c
            lax.fori_loop(0, VPT, body, 0)
            pltpu.sync_copy(av, alpha_out.at[h, pl.ds(w * EPT, EPT)])

    return k


def _sc_agg(H, NCH):
    @functools.partial(
        pl.kernel,
        out_type=jax.ShapeDtypeStruct((NC, NCH, N, CW), jnp.float32),
        mesh=plsc.VectorSubcoreMesh(**_MESH),
        scratch_types=[
            pltpu.VMEM((EPT,), jnp.int32),
            pltpu.VMEM((EPT,), jnp.int32),
            pltpu.VMEM((WPT, KW), jnp.int32),
            pltpu.VMEM((EPT,), jnp.float32),
            pltpu.VMEM((3, KW, CW), jnp.float32),
            pltpu.VMEM((128, CW), jnp.float32),
            pltpu.SemaphoreType.DMA,
            pltpu.SemaphoreType.DMA,
            pltpu.VMEM_SHARED((N, CW), jnp.float32),
        ],
        **_PARAMS,
    )
    def k(h_hbm, src_hbm, dst3_hbm, alpha_hbm, part_out,
          srcv, sidxv, dstv2d, alphav, rowbuf, zbuf, gsem, ssem, accum):
        cid = lax.axis_index("c")
        sid = lax.axis_index("s")
        w = cid * NS + sid
        r0 = sid * 624

        pltpu.sync_copy(src_hbm.at[w], srcv)
        pltpu.sync_copy(dst3_hbm.at[w], dstv2d)

        def zb(i, c):
            for j in range(CW // 16):
                zbuf[i, pl.ds(j * 16, 16)] = jnp.zeros((16,), jnp.float32)
            return c
        lax.fori_loop(0, 128, zb, 0)

        def gstart(wi):
            b = wi % 3
            pltpu.make_async_copy(
                h_hbm.at[sidxv.at[pl.ds(wi * KW, KW)]],
                rowbuf.at[b], gsem).start()

        def gwait():
            pltpu.make_async_copy(
                h_hbm.at[sidxv.at[pl.ds(0, KW)]],
                rowbuf.at[0], gsem).wait()

        def swait():
            pltpu.make_async_copy(
                rowbuf.at[0], accum.at[dstv2d.at[0]], ssem).wait()

        for ch in range(NCH):
            h = ch // (NCH // H)
            for b in range(5):
                pltpu.sync_copy(zbuf, accum.at[pl.ds(r0 + b * 128, 128)])
            pltpu.sync_copy(alpha_hbm.at[h, pl.ds(w * EPT, EPT)], alphav)

            def sbody(i, c, _ch=ch):
                sidxv[pl.ds(i * 16, 16)] = srcv[pl.ds(i * 16, 16)] + _ch * N
                return c
            lax.fori_loop(0, VPT, sbody, 0)
            plsc.subcore_barrier()

            gstart(0)

            def wbody(wi, cc):
                @pl.when(wi < WPT - 1)
                def _():
                    gstart(wi + 1)
                gwait()
                b = wi % 3

                def mb(g, c2):
                    a16 = alphav[pl.ds(wi * KW + g * 16, 16)]
                    for jj in range(16):
                        a = a16[jnp.full((16,), jj, jnp.int32)]
                        e = g * 16 + jj
                        for j in range(CW // 16):
                            sl = pl.ds(j * 16, 16)
                            rowbuf[b, e, sl] = rowbuf[b, e, sl] * a
                    return c2
                lax.fori_loop(0, KW // 16, mb, 0)

                @pl.when(wi > 0)
                def _():
                    swait()
                pltpu.async_copy(rowbuf.at[b], accum.at[dstv2d.at[wi]],
                                 ssem, add=True)
                return cc
            lax.fori_loop(0, WPT, wbody, 0)
            swait()
            plsc.subcore_barrier()
            pltpu.sync_copy(accum.at[pl.ds(r0, 640)],
                            part_out.at[cid, ch, pl.ds(r0, 640)])
            plsc.subcore_barrier()

    return k


def _tc_prep1(x, Ws, Wd, As16, Ad16):
    def body(x_ref, ws_ref, wd_ref, as_ref, ad_ref, hcm_ref, a16_ref):
        xb = x_ref[...]
        hs = jnp.dot(xb, ws_ref[...], preferred_element_type=jnp.float32)
        hd = jnp.dot(xb, wd_ref[...], preferred_element_type=jnp.float32)
        for cc in range(16):
            hcm_ref[cc] = hs[:, cc * CW:(cc + 1) * CW]
        a16_ref[...] = (jnp.dot(hs, as_ref[...], preferred_element_type=jnp.float32,
                                precision=lax.Precision.HIGHEST)
                        + jnp.dot(hd, ad_ref[...], preferred_element_type=jnp.float32,
                                  precision=lax.Precision.HIGHEST))

    return pl.pallas_call(
        body,
        grid=(10,),
        in_specs=[pl.BlockSpec((1000, D), lambda i: (i, 0)),
                  pl.BlockSpec((D, H1 * C1), lambda i: (0, 0)),
                  pl.BlockSpec((D, H1 * C1), lambda i: (0, 0)),
                  pl.BlockSpec((H1 * C1, 16), lambda i: (0, 0)),
                  pl.BlockSpec((H1 * C1, 16), lambda i: (0, 0))],
        out_specs=[pl.BlockSpec((16, 1000, CW), lambda i: (0, i, 0)),
                   pl.BlockSpec((1000, 16), lambda i: (i, 0))],
        out_shape=[jax.ShapeDtypeStruct((16, N, CW), jnp.float32),
                   jax.ShapeDtypeStruct((N, 16), jnp.float32)],
    )(x, Ws, Wd, As16, Ad16)


def _tc_prep2(part1, b1r, Ws2, Wd2, As16, Ad16):
    def body(p_ref, b_ref, ws_ref, wd_ref, as_ref, ad_ref, hcm_ref, a16_ref):
        h2 = jnp.concatenate(
            [p_ref[0, cc] + p_ref[1, cc] for cc in range(16)], axis=1)
        h2 = jnp.maximum(h2 + b_ref[...], 0.0)
        hs = jnp.dot(h2, ws_ref[...], preferred_element_type=jnp.float32)
        hd = jnp.dot(h2, wd_ref[...], preferred_element_type=jnp.float32)
        for cc in range(4):
            hcm_ref[cc] = hs[:, cc * CW:(cc + 1) * CW]
        a16_ref[...] = (jnp.dot(hs, as_ref[...], preferred_element_type=jnp.float32,
                                precision=lax.Precision.HIGHEST)
                        + jnp.dot(hd, ad_ref[...], preferred_element_type=jnp.float32,
                                  precision=lax.Precision.HIGHEST))

    return pl.pallas_call(
        body,
        grid=(10,),
        in_specs=[pl.BlockSpec((2, 16, 1000, CW), lambda i: (0, 0, i, 0)),
                  pl.BlockSpec((1, H1 * C1), lambda i: (0, 0)),
                  pl.BlockSpec((H1 * C1, C2), lambda i: (0, 0)),
                  pl.BlockSpec((H1 * C1, C2), lambda i: (0, 0)),
                  pl.BlockSpec((C2, 16), lambda i: (0, 0)),
                  pl.BlockSpec((C2, 16), lambda i: (0, 0))],
        out_specs=[pl.BlockSpec((4, 1000, CW), lambda i: (0, i, 0)),
                   pl.BlockSpec((1000, 16), lambda i: (i, 0))],
        out_shape=[jax.ShapeDtypeStruct((4, N, CW), jnp.float32),
                   jax.ShapeDtypeStruct((N, 16), jnp.float32)],
    )(part1, b1r, Ws2, Wd2, As16, Ad16)


def _tc_final(part2, b2r):
    def body(p_ref, b_ref, o_ref):
        o_ref[...] = jnp.concatenate(
            [p_ref[0, cc] + p_ref[1, cc] for cc in range(4)],
            axis=1) + b_ref[...]

    return pl.pallas_call(
        body,
        grid=(10,),
        in_specs=[pl.BlockSpec((2, 4, 1000, CW), lambda i: (0, 0, i, 0)),
                  pl.BlockSpec((1, C2), lambda i: (0, 0))],
        out_specs=pl.BlockSpec((1000, C2), lambda i: (i, 0)),
        out_shape=jax.ShapeDtypeStruct((N, C2), jnp.float32),
    )(part2, b2r)


def kernel(x, edge_index, W_src1, W_dst1, att_src1, att_dst1, b1,
           W_src2, W_dst2, att_src2, att_dst2, b2):
    f32 = jnp.float32
    src = edge_index[0].astype(jnp.int32)
    dst = edge_index[1].astype(jnp.int32)

    npad = EPAD - E
    padv = (jnp.arange(npad, dtype=jnp.int32) * 97) % N
    srcp = jnp.concatenate([src, padv]).reshape(NW, EPT)
    dstp_flat = jnp.concatenate([dst, padv])
    dstp = dstp_flat.reshape(NW, EPT)
    dst3 = dstp_flat.reshape(NW, WPT, KW)

    eye4 = jnp.eye(H1, dtype=f32)
    As1 = (att_src1[:, :, None] * eye4[:, None, :]).reshape(H1 * C1, H1)
    Ad1 = (att_dst1[:, :, None] * eye4[:, None, :]).reshape(H1 * C1, H1)
    As16_1 = jnp.pad(As1, ((0, 0), (0, 12)))
    Ad16_1 = jnp.pad(Ad1, ((0, 0), (4, 8)))
    As16_2 = jnp.pad(att_src2.T, ((0, 0), (0, 15)))
    Ad16_2 = jnp.pad(att_dst2.T, ((0, 0), (4, 11)))

    hcm, a16 = _tc_prep1(x, W_src1, W_dst1, As16_1, Ad16_1)
    ex1, den1 = _sc_edge(H1, 40960)(a16, srcp, dstp)
    al1 = _sc_alpha(H1, 40960)(ex1, dstp, den1)
    part1 = _sc_agg(H1, 16)(hcm.reshape(16 * N, CW), srcp, dst3, al1)

    h2cm, a16_2 = _tc_prep2(part1, b1.reshape(1, H1 * C1),
                            W_src2, W_dst2, As16_2, Ad16_2)
    ex2, den2 = _sc_edge(1, 10240)(a16_2, srcp, dstp)
    al2 = _sc_alpha(1, 10240)(ex2, dstp, den2)
    part2 = _sc_agg(1, 4)(h2cm.reshape(4 * N, CW), srcp, dst3, al2)

    return _tc_final(part2, b2.reshape(1, C2))


# speedup vs baseline: 1.8530x; 1.8530x over previous
import functools

import jax
import jax.numpy as jnp
from jax import lax
from jax.experimental import pallas as pl
from jax.experimental.pallas import tpu as pltpu
from jax.experimental.pallas import tpu_sc as plsc

N = 10000
E = 160000
D = 256
H1 = 4
C1 = 256
C2 = 256

NC = 2
NS = 16
NW = NC * NS
EPT = 5120
EPAD = NW * EPT
KW = 256
WPT = EPT // KW
VPT = EPT // 16
RPT = N // NS
CW = 64

_MESH = dict(core_axis_name="c", subcore_axis_name="s")
_PARAMS = dict(compiler_params=pltpu.CompilerParams(
    needs_layout_passes=False, use_tc_tiling_on_sc=False))


def _sc_edge(H, NHP):
    ZS = NHP // NS
    KD = 512
    NWD = EPT // KD

    @functools.partial(
        pl.kernel,
        out_type=(jax.ShapeDtypeStruct((H, EPAD), jnp.float32),
                  jax.ShapeDtypeStruct((NC, NHP), jnp.float32)),
        mesh=plsc.VectorSubcoreMesh(**_MESH),
        scratch_types=[
            pltpu.VMEM((EPT,), jnp.int32),
            pltpu.VMEM((EPT,), jnp.int32),
            pltpu.VMEM((H, EPT), jnp.float32),
            pltpu.VMEM((KD,), jnp.int32),
            pltpu.VMEM((KD, 16), jnp.float32),
            pltpu.VMEM((KD, 16), jnp.float32),
            pltpu.VMEM((KD, 16), jnp.float32),
            pltpu.VMEM((128, 16), jnp.float32),
            pltpu.VMEM((128,), jnp.float32),
            pltpu.SemaphoreType.DMA,
            pltpu.VMEM_SHARED((N, 16), jnp.float32),
            pltpu.VMEM_SHARED((NHP, 16), jnp.float32),
        ],
        **_PARAMS,
    )
    def k(a16_hbm, src_hbm, dst_hbm, ex_out, den_out,
          srcv, dstv, exv, didxb, exrows, arows_s, arows_d, cbuf, compb, sem,
          a16sp, den2d):
        cid = lax.axis_index("c")
        sid = lax.axis_index("s")
        w = cid * NS + sid
        lane = lax.iota(jnp.int32, 16)
        zeros16i = jnp.zeros((16,), jnp.int32)

        def zr(i, c):
            exrows[i, pl.ds(0, 16)] = jnp.zeros((16,), jnp.float32)
            return c
        lax.fori_loop(0, KD, zr, 0)
        for b in range(ZS // KD):
            pltpu.sync_copy(exrows, den2d.at[pl.ds(sid * ZS + b * KD, KD)])
        if ZS % KD:
            pltpu.sync_copy(exrows.at[pl.ds(0, ZS % KD)],
                            den2d.at[pl.ds(sid * ZS + (ZS // KD) * KD, ZS % KD)])
        nb0 = sid * 624
        pltpu.sync_copy(a16_hbm.at[pl.ds(nb0, KD)], arows_s)
        pltpu.sync_copy(arows_s, a16sp.at[pl.ds(nb0, KD)])
        pltpu.sync_copy(a16_hbm.at[pl.ds(nb0 + KD, 128)],
                        arows_s.at[pl.ds(0, 128)])
        pltpu.sync_copy(arows_s.at[pl.ds(0, 128)],
                        a16sp.at[pl.ds(nb0 + KD, 128)])
        pltpu.sync_copy(src_hbm.at[w], srcv)
        pltpu.sync_copy(dst_hbm.at[w], dstv)
        plsc.subcore_barrier()

        ebase = w * EPT

        def wloop(wi, c):
            off0 = wi * KD
            pltpu.async_copy(a16sp.at[srcv.at[pl.ds(off0, KD)]],
                             arows_s, sem).wait()
            pltpu.async_copy(a16sp.at[dstv.at[pl.ds(off0, KD)]],
                             arows_d, sem).wait()

            def body(i, c2):
                valid = (ebase + off0 + i * 16 + lane) < E
                rows = lane + i * 16
                for h in range(H):
                    av = plsc.load_gather(arows_s, [rows, zeros16i + h])
                    bv = plsc.load_gather(arows_d, [rows, zeros16i + (4 + h)])
                    e = av + bv
                    e = jnp.maximum(e, 0.2 * e)
                    ex = jnp.where(valid, jnp.exp(e), 0.0)
                    exv[h, pl.ds(off0 + i * 16, 16)] = ex
                return c2
            lax.fori_loop(0, KD // 16, body, 0)

            for h in range(H):
                def fill(i, c2, _h=h):
                    d16 = dstv[pl.ds(off0 + i * 16, 16)]
                    didxb[pl.ds(i * 16, 16)] = d16 * H + _h
                    ex16 = exv[_h, pl.ds(off0 + i * 16, 16)]
                    plsc.store_scatter(exrows, [lane + i * 16, zeros16i], ex16)
                    return c2
                lax.fori_loop(0, KD // 16, fill, 0)
                pltpu.sync_copy(exrows, den2d.at[didxb], add=True)
            return c
        lax.fori_loop(0, NWD, wloop, 0)

        for h in range(H):
            pltpu.sync_copy(exv.at[h], ex_out.at[h, pl.ds(ebase, EPT)])

        plsc.subcore_barrier()

        def comp(t, c):
            base = sid * ZS + t * 128
            pltpu.sync_copy(den2d.at[pl.ds(base, 128)], cbuf)

            def gat(j, c2):
                g = plsc.load_gather(cbuf, [lane + j * 16, zeros16i])
                compb[pl.ds(j * 16, 16)] = g
                return c2
            lax.fori_loop(0, 8, gat, 0)
            pltpu.sync_copy(compb, den_out.at[cid, pl.ds(base, 128)])
            return c
        lax.fori_loop(0, ZS // 128, comp, 0)

    return k


def _sc_alpha(H, NHP):
    @functools.partial(
        pl.kernel,
        out_type=jax.ShapeDtypeStruct((H, EPAD), jnp.float32),
        mesh=plsc.VectorSubcoreMesh(**_MESH),
        scratch_types=[
            pltpu.VMEM((NHP,), jnp.float32),
            pltpu.VMEM((2048,), jnp.float32),
            pltpu.VMEM((EPT,), jnp.int32),
            pltpu.VMEM((EPT,), jnp.float32),
            pltpu.VMEM((EPT,), jnp.float32),
        ],
        **_PARAMS,
    )
    def k(ex_hbm, dst_hbm, den_hbm, alpha_out, denl, tmp, dstv, exv, av):
        cid = lax.axis_index("c")
        sid = lax.axis_index("s")
        w = cid * NS + sid

        pltpu.sync_copy(den_hbm.at[0], denl)

        def mbody(kk, c):
            pltpu.sync_copy(den_hbm.at[1, pl.ds(kk * 2048, 2048)], tmp)

            def abody(j, c2):
                sl = pl.ds(kk * 2048 + j * 16, 16)
                denl[sl] = denl[sl] + tmp[pl.ds(j * 16, 16)] + 1e-16
                return c2
            lax.fori_loop(0, 128, abody, 0)
            return c
        lax.fori_loop(0, NHP // 2048, mbody, 0)

        pltpu.sync_copy(dst_hbm.at[w], dstv)
        for h in range(H):
            pltpu.sync_copy(ex_hbm.at[h, pl.ds(w * EPT, EPT)], exv)

            def body(i, c, _h=h):
                d16 = dstv[pl.ds(i * 16, 16)]
                dv = plsc.load_gather(denl, [d16 * H + _h])
                av[pl.ds(i * 16, 16)] = exv[pl.ds(i * 16, 16)] / dv
                return c
            lax.fori_loop(0, VPT, body, 0)
            pltpu.sync_copy(av, alpha_out.at[h, pl.ds(w * EPT, EPT)])

    return k


def _sc_agg(H, NCH):
    @functools.partial(
        pl.kernel,
        out_type=jax.ShapeDtypeStruct((NC, NCH, N, CW), jnp.float32),
        mesh=plsc.VectorSubcoreMesh(**_MESH),
        scratch_types=[
            pltpu.VMEM((EPT,), jnp.int32),
            pltpu.VMEM((EPT,), jnp.int32),
            pltpu.VMEM((WPT, KW), jnp.int32),
            pltpu.VMEM((EPT,), jnp.float32),
            pltpu.VMEM((3, KW, CW), jnp.float32),
            pltpu.VMEM((128, CW), jnp.float32),
            pltpu.SemaphoreType.DMA,
            pltpu.SemaphoreType.DMA,
            pltpu.VMEM_SHARED((N, CW), jnp.float32),
        ],
        **_PARAMS,
    )
    def k(h_hbm, src_hbm, dst3_hbm, alpha_hbm, part_out,
          srcv, sidxv, dstv2d, alphav, rowbuf, zbuf, gsem, ssem, accum):
        cid = lax.axis_index("c")
        sid = lax.axis_index("s")
        w = cid * NS + sid
        r0 = sid * 624

        pltpu.sync_copy(src_hbm.at[w], srcv)
        pltpu.sync_copy(dst3_hbm.at[w], dstv2d)

        def zb(i, c):
            for j in range(CW // 16):
                zbuf[i, pl.ds(j * 16, 16)] = jnp.zeros((16,), jnp.float32)
            return c
        lax.fori_loop(0, 128, zb, 0)

        def gstart(wi):
            b = wi % 3
            pltpu.make_async_copy(
                h_hbm.at[sidxv.at[pl.ds(wi * KW, KW)]],
                rowbuf.at[b], gsem).start()

        def gwait():
            pltpu.make_async_copy(
                h_hbm.at[sidxv.at[pl.ds(0, KW)]],
                rowbuf.at[0], gsem).wait()

        def swait():
            pltpu.make_async_copy(
                rowbuf.at[0], accum.at[dstv2d.at[0]], ssem).wait()

        for ch in range(NCH):
            h = ch // (NCH // H)
            for b in range(5):
                pltpu.sync_copy(zbuf, accum.at[pl.ds(r0 + b * 128, 128)])
            pltpu.sync_copy(alpha_hbm.at[h, pl.ds(w * EPT, EPT)], alphav)

            def sbody(i, c, _ch=ch):
                sidxv[pl.ds(i * 16, 16)] = srcv[pl.ds(i * 16, 16)] + _ch * N
                return c
            lax.fori_loop(0, VPT, sbody, 0)
            plsc.subcore_barrier()

            gstart(0)

            def wbody(wi, cc):
                @pl.when(wi < WPT - 1)
                def _():
                    gstart(wi + 1)
                gwait()
                b = wi % 3

                def mb(g, c2):
                    a16 = alphav[pl.ds(wi * KW + g * 16, 16)]
                    for jj in range(16):
                        a = a16[jnp.full((16,), jj, jnp.int32)]
                        e = g * 16 + jj
                        for j in range(CW // 16):
                            sl = pl.ds(j * 16, 16)
                            rowbuf[b, e, sl] = rowbuf[b, e, sl] * a
                    return c2
                plsc.parallel_loop(0, KW // 16, 1, carry=jnp.int32(0))(mb)

                @pl.when(wi > 0)
                def _():
                    swait()
                pltpu.async_copy(rowbuf.at[b], accum.at[dstv2d.at[wi]],
                                 ssem, add=True)
                return cc
            lax.fori_loop(0, WPT, wbody, 0)
            swait()
            plsc.subcore_barrier()
            pltpu.sync_copy(accum.at[pl.ds(r0, 640)],
                            part_out.at[cid, ch, pl.ds(r0, 640)])
            plsc.subcore_barrier()

    return k


def _tc_prep1(x, Ws, Wd, As16, Ad16):
    def body(x_ref, ws_ref, wd_ref, as_ref, ad_ref, hcm_ref, a16_ref):
        xb = x_ref[...]
        hs = jnp.dot(xb, ws_ref[...], preferred_element_type=jnp.float32)
        hd = jnp.dot(xb, wd_ref[...], preferred_element_type=jnp.float32)
        for cc in range(16):
            hcm_ref[cc] = hs[:, cc * CW:(cc + 1) * CW]
        a16_ref[...] = (jnp.dot(hs, as_ref[...], preferred_element_type=jnp.float32,
                                precision=lax.Precision.HIGHEST)
                        + jnp.dot(hd, ad_ref[...], preferred_element_type=jnp.float32,
                                  precision=lax.Precision.HIGHEST))

    return pl.pallas_call(
        body,
        grid=(10,),
        in_specs=[pl.BlockSpec((1000, D), lambda i: (i, 0)),
                  pl.BlockSpec((D, H1 * C1), lambda i: (0, 0)),
                  pl.BlockSpec((D, H1 * C1), lambda i: (0, 0)),
                  pl.BlockSpec((H1 * C1, 16), lambda i: (0, 0)),
                  pl.BlockSpec((H1 * C1, 16), lambda i: (0, 0))],
        out_specs=[pl.BlockSpec((16, 1000, CW), lambda i: (0, i, 0)),
                   pl.BlockSpec((1000, 16), lambda i: (i, 0))],
        out_shape=[jax.ShapeDtypeStruct((16, N, CW), jnp.float32),
                   jax.ShapeDtypeStruct((N, 16), jnp.float32)],
    )(x, Ws, Wd, As16, Ad16)


def _tc_prep2(part1, b1r, Ws2, Wd2, As16, Ad16):
    def body(p_ref, b_ref, ws_ref, wd_ref, as_ref, ad_ref, hcm_ref, a16_ref):
        h2 = jnp.concatenate(
            [p_ref[0, cc] + p_ref[1, cc] for cc in range(16)], axis=1)
        h2 = jnp.maximum(h2 + b_ref[...], 0.0)
        hs = jnp.dot(h2, ws_ref[...], preferred_element_type=jnp.float32)
        hd = jnp.dot(h2, wd_ref[...], preferred_element_type=jnp.float32)
        for cc in range(4):
            hcm_ref[cc] = hs[:, cc * CW:(cc + 1) * CW]
        a16_ref[...] = (jnp.dot(hs, as_ref[...], preferred_element_type=jnp.float32,
                                precision=lax.Precision.HIGHEST)
                        + jnp.dot(hd, ad_ref[...], preferred_element_type=jnp.float32,
                                  precision=lax.Precision.HIGHEST))

    return pl.pallas_call(
        body,
        grid=(10,),
        in_specs=[pl.BlockSpec((2, 16, 1000, CW), lambda i: (0, 0, i, 0)),
                  pl.BlockSpec((1, H1 * C1), lambda i: (0, 0)),
                  pl.BlockSpec((H1 * C1, C2), lambda i: (0, 0)),
                  pl.BlockSpec((H1 * C1, C2), lambda i: (0, 0)),
                  pl.BlockSpec((C2, 16), lambda i: (0, 0)),
                  pl.BlockSpec((C2, 16), lambda i: (0, 0))],
        out_specs=[pl.BlockSpec((4, 1000, CW), lambda i: (0, i, 0)),
                   pl.BlockSpec((1000, 16), lambda i: (i, 0))],
        out_shape=[jax.ShapeDtypeStruct((4, N, CW), jnp.float32),
                   jax.ShapeDtypeStruct((N, 16), jnp.float32)],
    )(part1, b1r, Ws2, Wd2, As16, Ad16)


def _tc_final(part2, b2r):
    def body(p_ref, b_ref, o_ref):
        o_ref[...] = jnp.concatenate(
            [p_ref[0, cc] + p_ref[1, cc] for cc in range(4)],
            axis=1) + b_ref[...]

    return pl.pallas_call(
        body,
        grid=(10,),
        in_specs=[pl.BlockSpec((2, 4, 1000, CW), lambda i: (0, 0, i, 0)),
                  pl.BlockSpec((1, C2), lambda i: (0, 0))],
        out_specs=pl.BlockSpec((1000, C2), lambda i: (i, 0)),
        out_shape=jax.ShapeDtypeStruct((N, C2), jnp.float32),
    )(part2, b2r)


def kernel(x, edge_index, W_src1, W_dst1, att_src1, att_dst1, b1,
           W_src2, W_dst2, att_src2, att_dst2, b2):
    f32 = jnp.float32
    src = edge_index[0].astype(jnp.int32)
    dst = edge_index[1].astype(jnp.int32)

    npad = EPAD - E
    padv = (jnp.arange(npad, dtype=jnp.int32) * 97) % N
    srcp = jnp.concatenate([src, padv]).reshape(NW, EPT)
    dstp_flat = jnp.concatenate([dst, padv])
    dstp = dstp_flat.reshape(NW, EPT)
    dst3 = dstp_flat.reshape(NW, WPT, KW)

    eye4 = jnp.eye(H1, dtype=f32)
    As1 = (att_src1[:, :, None] * eye4[:, None, :]).reshape(H1 * C1, H1)
    Ad1 = (att_dst1[:, :, None] * eye4[:, None, :]).reshape(H1 * C1, H1)
    As16_1 = jnp.pad(As1, ((0, 0), (0, 12)))
    Ad16_1 = jnp.pad(Ad1, ((0, 0), (4, 8)))
    As16_2 = jnp.pad(att_src2.T, ((0, 0), (0, 15)))
    Ad16_2 = jnp.pad(att_dst2.T, ((0, 0), (4, 11)))

    hcm, a16 = _tc_prep1(x, W_src1, W_dst1, As16_1, Ad16_1)
    ex1, den1 = _sc_edge(H1, 40960)(a16, srcp, dstp)
    al1 = _sc_alpha(H1, 40960)(ex1, dstp, den1)
    part1 = _sc_agg(H1, 16)(hcm.reshape(16 * N, CW), srcp, dst3, al1)

    h2cm, a16_2 = _tc_prep2(part1, b1.reshape(1, H1 * C1),
                            W_src2, W_dst2, As16_2, Ad16_2)
    ex2, den2 = _sc_edge(1, 10240)(a16_2, srcp, dstp)
    al2 = _sc_alpha(1, 10240)(ex2, dstp, den2)
    part2 = _sc_agg(1, 4)(h2cm.reshape(4 * N, CW), srcp, dst3, al2)

    return _tc_final(part2, b2.reshape(1, C2))


# speedup vs baseline: 1.9516x; 1.0533x over previous
import functools

import jax
import jax.numpy as jnp
from jax import lax
from jax.experimental import pallas as pl
from jax.experimental.pallas import tpu as pltpu
from jax.experimental.pallas import tpu_sc as plsc

N = 10000
E = 160000
D = 256
H1 = 4
C1 = 256
C2 = 256

NC = 2
NS = 16
NW = NC * NS
EPT = 5120
EPAD = NW * EPT
KW = 256
WPT = EPT // KW
VPT = EPT // 16
RPT = N // NS
CW = 64

_MESH = dict(core_axis_name="c", subcore_axis_name="s")
_PARAMS = dict(compiler_params=pltpu.CompilerParams(
    needs_layout_passes=False, use_tc_tiling_on_sc=False))


def _sc_edge(H, NHP):
    ZS = NHP // NS
    KD = 512
    NWD = EPT // KD

    @functools.partial(
        pl.kernel,
        out_type=(jax.ShapeDtypeStruct((H, EPAD), jnp.float32),
                  jax.ShapeDtypeStruct((NC, NHP), jnp.float32)),
        mesh=plsc.VectorSubcoreMesh(**_MESH),
        scratch_types=[
            pltpu.VMEM((EPT,), jnp.int32),
            pltpu.VMEM((EPT,), jnp.int32),
            pltpu.VMEM((H, EPT), jnp.float32),
            pltpu.VMEM((KD,), jnp.int32),
            pltpu.VMEM((KD, 16), jnp.float32),
            pltpu.VMEM((KD, 16), jnp.float32),
            pltpu.VMEM((KD, 16), jnp.float32),
            pltpu.VMEM((128, 16), jnp.float32),
            pltpu.VMEM((128,), jnp.float32),
            pltpu.SemaphoreType.DMA,
            pltpu.VMEM_SHARED((N, 16), jnp.float32),
            pltpu.VMEM_SHARED((NHP, 16), jnp.float32),
        ],
        **_PARAMS,
    )
    def k(a16_hbm, src_hbm, dst_hbm, ex_out, den_out,
          srcv, dstv, exv, didxb, exrows, arows_s, arows_d, cbuf, compb, sem,
          a16sp, den2d):
        cid = lax.axis_index("c")
        sid = lax.axis_index("s")
        w = cid * NS + sid
        lane = lax.iota(jnp.int32, 16)
        zeros16i = jnp.zeros((16,), jnp.int32)

        def zr(i, c):
            exrows[i, pl.ds(0, 16)] = jnp.zeros((16,), jnp.float32)
            return c
        lax.fori_loop(0, KD, zr, 0)
        for b in range(ZS // KD):
            pltpu.sync_copy(exrows, den2d.at[pl.ds(sid * ZS + b * KD, KD)])
        if ZS % KD:
            pltpu.sync_copy(exrows.at[pl.ds(0, ZS % KD)],
                            den2d.at[pl.ds(sid * ZS + (ZS // KD) * KD, ZS % KD)])
        nb0 = sid * 624
        pltpu.sync_copy(a16_hbm.at[pl.ds(nb0, KD)], arows_s)
        pltpu.sync_copy(arows_s, a16sp.at[pl.ds(nb0, KD)])
        pltpu.sync_copy(a16_hbm.at[pl.ds(nb0 + KD, 128)],
                        arows_s.at[pl.ds(0, 128)])
        pltpu.sync_copy(arows_s.at[pl.ds(0, 128)],
                        a16sp.at[pl.ds(nb0 + KD, 128)])
        pltpu.sync_copy(src_hbm.at[w], srcv)
        pltpu.sync_copy(dst_hbm.at[w], dstv)
        plsc.subcore_barrier()

        ebase = w * EPT

        def wloop(wi, c):
            off0 = wi * KD
            pltpu.async_copy(a16sp.at[srcv.at[pl.ds(off0, KD)]],
                             arows_s, sem).wait()
            pltpu.async_copy(a16sp.at[dstv.at[pl.ds(off0, KD)]],
                             arows_d, sem).wait()

            def body(i, c2):
                valid = (ebase + off0 + i * 16 + lane) < E
                rows = lane + i * 16
                for h in range(H):
                    av = plsc.load_gather(arows_s, [rows, zeros16i + h])
                    bv = plsc.load_gather(arows_d, [rows, zeros16i + (4 + h)])
                    e = av + bv
                    e = jnp.maximum(e, 0.2 * e)
                    ex = jnp.where(valid, jnp.exp(e), 0.0)
                    exv[h, pl.ds(off0 + i * 16, 16)] = ex
                return c2
            plsc.parallel_loop(0, KD // 16, 1, carry=jnp.int32(0))(body)

            for h in range(H):
                def fill(i, c2, _h=h):
                    d16 = dstv[pl.ds(off0 + i * 16, 16)]
                    didxb[pl.ds(i * 16, 16)] = d16 * H + _h
                    ex16 = exv[_h, pl.ds(off0 + i * 16, 16)]
                    plsc.store_scatter(exrows, [lane + i * 16, zeros16i], ex16)
                    return c2
                plsc.parallel_loop(0, KD // 16, 1, carry=jnp.int32(0))(fill)
                pltpu.sync_copy(exrows, den2d.at[didxb], add=True)
            return c
        lax.fori_loop(0, NWD, wloop, 0)

        for h in range(H):
            pltpu.sync_copy(exv.at[h], ex_out.at[h, pl.ds(ebase, EPT)])

        plsc.subcore_barrier()

        def comp(t, c):
            base = sid * ZS + t * 128
            pltpu.sync_copy(den2d.at[pl.ds(base, 128)], cbuf)

            def gat(j, c2):
                g = plsc.load_gather(cbuf, [lane + j * 16, zeros16i])
                compb[pl.ds(j * 16, 16)] = g
                return c2
            plsc.parallel_loop(0, 8, 1, carry=jnp.int32(0))(gat)
            pltpu.sync_copy(compb, den_out.at[cid, pl.ds(base, 128)])
            return c
        lax.fori_loop(0, ZS // 128, comp, 0)

    return k


def _sc_alpha(H, NHP):
    @functools.partial(
        pl.kernel,
        out_type=jax.ShapeDtypeStruct((H, EPAD), jnp.float32),
        mesh=plsc.VectorSubcoreMesh(**_MESH),
        scratch_types=[
            pltpu.VMEM((NHP,), jnp.float32),
            pltpu.VMEM((2048,), jnp.float32),
            pltpu.VMEM((EPT,), jnp.int32),
            pltpu.VMEM((EPT,), jnp.float32),
            pltpu.VMEM((EPT,), jnp.float32),
        ],
        **_PARAMS,
    )
    def k(ex_hbm, dst_hbm, den_hbm, alpha_out, denl, tmp, dstv, exv, av):
        cid = lax.axis_index("c")
        sid = lax.axis_index("s")
        w = cid * NS + sid

        pltpu.sync_copy(den_hbm.at[0], denl)

        def mbody(kk, c):
            pltpu.sync_copy(den_hbm.at[1, pl.ds(kk * 2048, 2048)], tmp)

            def abody(j, c2):
                sl = pl.ds(kk * 2048 + j * 16, 16)
                denl[sl] = denl[sl] + tmp[pl.ds(j * 16, 16)] + 1e-16
                return c2
            lax.fori_loop(0, 128, abody, 0)
            return c
        lax.fori_loop(0, NHP // 2048, mbody, 0)

        pltpu.sync_copy(dst_hbm.at[w], dstv)
        for h in range(H):
            pltpu.sync_copy(ex_hbm.at[h, pl.ds(w * EPT, EPT)], exv)

            def body(i, c, _h=h):
                d16 = dstv[pl.ds(i * 16, 16)]
                dv = plsc.load_gather(denl, [d16 * H + _h])
                av[pl.ds(i * 16, 16)] = exv[pl.ds(i * 16, 16)] / dv
                return c
            plsc.parallel_loop(0, VPT, 1, carry=jnp.int32(0))(body)
            pltpu.sync_copy(av, alpha_out.at[h, pl.ds(w * EPT, EPT)])

    return k


def _sc_agg(H, NCH):
    @functools.partial(
        pl.kernel,
        out_type=jax.ShapeDtypeStruct((NC, NCH, N, CW), jnp.float32),
        mesh=plsc.VectorSubcoreMesh(**_MESH),
        scratch_types=[
            pltpu.VMEM((EPT,), jnp.int32),
            pltpu.VMEM((EPT,), jnp.int32),
            pltpu.VMEM((WPT, KW), jnp.int32),
            pltpu.VMEM((EPT,), jnp.float32),
            pltpu.VMEM((3, KW, CW), jnp.float32),
            pltpu.VMEM((128, CW), jnp.float32),
            pltpu.SemaphoreType.DMA,
            pltpu.SemaphoreType.DMA,
            pltpu.VMEM_SHARED((N, CW), jnp.float32),
        ],
        **_PARAMS,
    )
    def k(h_hbm, src_hbm, dst3_hbm, alpha_hbm, part_out,
          srcv, sidxv, dstv2d, alphav, rowbuf, zbuf, gsem, ssem, accum):
        cid = lax.axis_index("c")
        sid = lax.axis_index("s")
        w = cid * NS + sid
        r0 = sid * 624

        pltpu.sync_copy(src_hbm.at[w], srcv)
        pltpu.sync_copy(dst3_hbm.at[w], dstv2d)

        def zb(i, c):
            for j in range(CW // 16):
                zbuf[i, pl.ds(j * 16, 16)] = jnp.zeros((16,), jnp.float32)
            return c
        lax.fori_loop(0, 128, zb, 0)

        def gstart(wi):
            b = wi % 3
            pltpu.make_async_copy(
                h_hbm.at[sidxv.at[pl.ds(wi * KW, KW)]],
                rowbuf.at[b], gsem).start()

        def gwait():
            pltpu.make_async_copy(
                h_hbm.at[sidxv.at[pl.ds(0, KW)]],
                rowbuf.at[0], gsem).wait()

        def swait():
            pltpu.make_async_copy(
                rowbuf.at[0], accum.at[dstv2d.at[0]], ssem).wait()

        for ch in range(NCH):
            h = ch // (NCH // H)
            for b in range(5):
                pltpu.make_async_copy(
                    zbuf, accum.at[pl.ds(r0 + b * 128, 128)], ssem).start()
            pltpu.sync_copy(alpha_hbm.at[h, pl.ds(w * EPT, EPT)], alphav)

            def sbody(i, c, _ch=ch):
                sidxv[pl.ds(i * 16, 16)] = srcv[pl.ds(i * 16, 16)] + _ch * N
                return c
            plsc.parallel_loop(0, VPT, 1, carry=jnp.int32(0))(sbody)
            for b in range(5):
                pltpu.make_async_copy(
                    zbuf, accum.at[pl.ds(r0, 128)], ssem).wait()
            plsc.subcore_barrier()

            gstart(0)

            def wbody(wi, cc):
                @pl.when(wi < WPT - 1)
                def _():
                    gstart(wi + 1)
                gwait()
                b = wi % 3

                def mb(g, c2):
                    a16 = alphav[pl.ds(wi * KW + g * 16, 16)]
                    for jj in range(16):
                        a = a16[jnp.full((16,), jj, jnp.int32)]
                        e = g * 16 + jj
                        for j in range(CW // 16):
                            sl = pl.ds(j * 16, 16)
                            rowbuf[b, e, sl] = rowbuf[b, e, sl] * a
                    return c2
                plsc.parallel_loop(0, KW // 16, 1, carry=jnp.int32(0))(mb)

                @pl.when(wi > 0)
                def _():
                    swait()
                pltpu.async_copy(rowbuf.at[b], accum.at[dstv2d.at[wi]],
                                 ssem, add=True)
                return cc
            lax.fori_loop(0, WPT, wbody, 0)
            swait()
            plsc.subcore_barrier()
            pltpu.sync_copy(accum.at[pl.ds(r0, 640)],
                            part_out.at[cid, ch, pl.ds(r0, 640)])
            plsc.subcore_barrier()

    return k


def _tc_prep1(x, Ws, Wd, As16, Ad16):
    def body(x_ref, ws_ref, wd_ref, as_ref, ad_ref, hcm_ref, a16_ref):
        xb = x_ref[...]
        hs = jnp.dot(xb, ws_ref[...], preferred_element_type=jnp.float32)
        hd = jnp.dot(xb, wd_ref[...], preferred_element_type=jnp.float32)
        for cc in range(16):
            hcm_ref[cc] = hs[:, cc * CW:(cc + 1) * CW]
        a16_ref[...] = (jnp.dot(hs, as_ref[...], preferred_element_type=jnp.float32,
                                precision=lax.Precision.HIGHEST)
                        + jnp.dot(hd, ad_ref[...], preferred_element_type=jnp.float32,
                                  precision=lax.Precision.HIGHEST))

    return pl.pallas_call(
        body,
        grid=(10,),
        in_specs=[pl.BlockSpec((1000, D), lambda i: (i, 0)),
                  pl.BlockSpec((D, H1 * C1), lambda i: (0, 0)),
                  pl.BlockSpec((D, H1 * C1), lambda i: (0, 0)),
                  pl.BlockSpec((H1 * C1, 16), lambda i: (0, 0)),
                  pl.BlockSpec((H1 * C1, 16), lambda i: (0, 0))],
        out_specs=[pl.BlockSpec((16, 1000, CW), lambda i: (0, i, 0)),
                   pl.BlockSpec((1000, 16), lambda i: (i, 0))],
        out_shape=[jax.ShapeDtypeStruct((16, N, CW), jnp.float32),
                   jax.ShapeDtypeStruct((N, 16), jnp.float32)],
    )(x, Ws, Wd, As16, Ad16)


def _tc_prep2(part1, b1r, Ws2, Wd2, As16, Ad16):
    def body(p_ref, b_ref, ws_ref, wd_ref, as_ref, ad_ref, hcm_ref, a16_ref):
        h2 = jnp.concatenate(
            [p_ref[0, cc] + p_ref[1, cc] for cc in range(16)], axis=1)
        h2 = jnp.maximum(h2 + b_ref[...], 0.0)
        hs = jnp.dot(h2, ws_ref[...], preferred_element_type=jnp.float32)
        hd = jnp.dot(h2, wd_ref[...], preferred_element_type=jnp.float32)
        for cc in range(4):
            hcm_ref[cc] = hs[:, cc * CW:(cc + 1) * CW]
        a16_ref[...] = (jnp.dot(hs, as_ref[...], preferred_element_type=jnp.float32,
                                precision=lax.Precision.HIGHEST)
                        + jnp.dot(hd, ad_ref[...], preferred_element_type=jnp.float32,
                                  precision=lax.Precision.HIGHEST))

    return pl.pallas_call(
        body,
        grid=(10,),
        in_specs=[pl.BlockSpec((2, 16, 1000, CW), lambda i: (0, 0, i, 0)),
                  pl.BlockSpec((1, H1 * C1), lambda i: (0, 0)),
                  pl.BlockSpec((H1 * C1, C2), lambda i: (0, 0)),
                  pl.BlockSpec((H1 * C1, C2), lambda i: (0, 0)),
                  pl.BlockSpec((C2, 16), lambda i: (0, 0)),
                  pl.BlockSpec((C2, 16), lambda i: (0, 0))],
        out_specs=[pl.BlockSpec((4, 1000, CW), lambda i: (0, i, 0)),
                   pl.BlockSpec((1000, 16), lambda i: (i, 0))],
        out_shape=[jax.ShapeDtypeStruct((4, N, CW), jnp.float32),
                   jax.ShapeDtypeStruct((N, 16), jnp.float32)],
    )(part1, b1r, Ws2, Wd2, As16, Ad16)


def _tc_final(part2, b2r):
    def body(p_ref, b_ref, o_ref):
        o_ref[...] = jnp.concatenate(
            [p_ref[0, cc] + p_ref[1, cc] for cc in range(4)],
            axis=1) + b_ref[...]

    return pl.pallas_call(
        body,
        grid=(10,),
        in_specs=[pl.BlockSpec((2, 4, 1000, CW), lambda i: (0, 0, i, 0)),
                  pl.BlockSpec((1, C2), lambda i: (0, 0))],
        out_specs=pl.BlockSpec((1000, C2), lambda i: (i, 0)),
        out_shape=jax.ShapeDtypeStruct((N, C2), jnp.float32),
    )(part2, b2r)


def kernel(x, edge_index, W_src1, W_dst1, att_src1, att_dst1, b1,
           W_src2, W_dst2, att_src2, att_dst2, b2):
    f32 = jnp.float32
    src = edge_index[0].astype(jnp.int32)
    dst = edge_index[1].astype(jnp.int32)

    npad = EPAD - E
    padv = (jnp.arange(npad, dtype=jnp.int32) * 97) % N
    srcp = jnp.concatenate([src, padv]).reshape(NW, EPT)
    dstp_flat = jnp.concatenate([dst, padv])
    dstp = dstp_flat.reshape(NW, EPT)
    dst3 = dstp_flat.reshape(NW, WPT, KW)

    eye4 = jnp.eye(H1, dtype=f32)
    As1 = (att_src1[:, :, None] * eye4[:, None, :]).reshape(H1 * C1, H1)
    Ad1 = (att_dst1[:, :, None] * eye4[:, None, :]).reshape(H1 * C1, H1)
    As16_1 = jnp.pad(As1, ((0, 0), (0, 12)))
    Ad16_1 = jnp.pad(Ad1, ((0, 0), (4, 8)))
    As16_2 = jnp.pad(att_src2.T, ((0, 0), (0, 15)))
    Ad16_2 = jnp.pad(att_dst2.T, ((0, 0), (4, 11)))

    hcm, a16 = _tc_prep1(x, W_src1, W_dst1, As16_1, Ad16_1)
    ex1, den1 = _sc_edge(H1, 40960)(a16, srcp, dstp)
    al1 = _sc_alpha(H1, 40960)(ex1, dstp, den1)
    part1 = _sc_agg(H1, 16)(hcm.reshape(16 * N, CW), srcp, dst3, al1)

    h2cm, a16_2 = _tc_prep2(part1, b1.reshape(1, H1 * C1),
                            W_src2, W_dst2, As16_2, Ad16_2)
    ex2, den2 = _sc_edge(1, 10240)(a16_2, srcp, dstp)
    al2 = _sc_alpha(1, 10240)(ex2, dstp, den2)
    part2 = _sc_agg(1, 4)(h2cm.reshape(4 * N, CW), srcp, dst3, al2)

    return _tc_final(part2, b2.reshape(1, C2))
